# baseline (speedup 1.0000x reference)
import functools

import jax
import jax.numpy as jnp
from jax import lax
from jax.experimental import pallas as pl
from jax.experimental.pallas import tpu as pltpu
from jax.experimental.pallas import tpu_sc as plsc

_NUM_EMB = 8192
_DIM = 256
_COMMIT = 0.25
_T = 1024


def _dist_argmin_kernel(x_ref, e_ref, idx_ref, loss_ref, b_ref):
    i = pl.program_id(0)

    @pl.when(i == 0)
    def _():
        e0 = e_ref[...]
        b_ref[...] = lax.dot_general(
            jnp.ones((1, _DIM), jnp.float32), e0 * e0,
            (((1,), (1,)), ((), ())))
        loss_ref[0, 0] = 0.0

    x = x_ref[...]
    e = e_ref[...]

    a = jnp.sum(x * x, axis=1, keepdims=True)
    m = lax.dot_general(x, e, (((1,), (1,)), ((), ())))

    d = a + b_ref[0:1, :] - 2.0 * m

    dmin = jnp.min(d, axis=1, keepdims=True)
    iotaf = lax.broadcasted_iota(jnp.int32, d.shape, 1).astype(jnp.float32)
    idxf = jnp.min(jnp.where(d == dmin, iotaf, float(_NUM_EMB)), axis=1)
    idx_ref[...] = idxf.astype(jnp.int32)[None, None, :]
    loss_ref[0, 0] += jnp.sum(dmin)


@functools.lru_cache(maxsize=4)
def _make_sc_gather(n_rows):
    info = plsc.get_sparse_core_info()
    n_cores = info.num_cores
    rows_per_w = n_rows // (n_cores * info.num_subcores)

    def _sc_gather_kernel(table_hbm, idx_hbm, out_hbm, idx_v, rows_v, sem):
        wid = lax.axis_index("s") * n_cores + lax.axis_index("c")
        base = wid * rows_per_w
        pltpu.sync_copy(idx_hbm.at[pl.ds(base, rows_per_w)], idx_v)
        pltpu.async_copy(table_hbm.at[idx_v], rows_v, sem).wait()
        pltpu.sync_copy(rows_v, out_hbm.at[pl.ds(base, rows_per_w)])

    return functools.partial(
        pl.kernel,
        out_type=jax.ShapeDtypeStruct((n_rows, _DIM), jnp.float32),
        mesh=plsc.VectorSubcoreMesh(core_axis_name="c", subcore_axis_name="s"),
        scratch_types=[
            pltpu.VMEM((rows_per_w,), jnp.int32),
            pltpu.VMEM((rows_per_w, _DIM), jnp.float32),
            pltpu.SemaphoreType.DMA,
        ],
    )(_sc_gather_kernel)


_N_SPLIT = 2


def _dist_argmin(flat_chunk):
    n_tok = flat_chunk.shape[0]
    return pl.pallas_call(
        _dist_argmin_kernel,
        grid=(n_tok // _T,),
        in_specs=[
            pl.BlockSpec((_T, _DIM), lambda i: (i, 0)),
            pl.BlockSpec((_NUM_EMB, _DIM), lambda i: (0, 0)),
        ],
        out_specs=[
            pl.BlockSpec((1, 1, _T), lambda i: (i, 0, 0)),
            pl.BlockSpec(memory_space=pltpu.SMEM),
        ],
        out_shape=[
            jax.ShapeDtypeStruct((n_tok // _T, 1, _T), jnp.int32),
            jax.ShapeDtypeStruct((1, 1), jnp.float32),
        ],
        scratch_shapes=[pltpu.VMEM((1, _NUM_EMB), jnp.float32)],
    )


def kernel(inputs, emb_weight):
    B, C, H, W = inputs.shape
    n_tok = B * H * W
    n_chunk = n_tok // _N_SPLIT
    flat = jnp.transpose(inputs, (0, 2, 3, 1)).reshape(n_tok, _DIM)

    idx_parts, q_parts, loss_parts = [], [], []
    for s in range(_N_SPLIT):
        flat_s = lax.slice_in_dim(flat, s * n_chunk, (s + 1) * n_chunk)
        idx3, loss_sum = _dist_argmin(flat_s)(flat_s, emb_weight)
        idx_flat = idx3.reshape(n_chunk)
        q_parts.append(_make_sc_gather(n_chunk)(emb_weight, idx_flat))
        idx_parts.append(idx_flat)
        loss_parts.append(loss_sum[0, 0])

    idx_all = jnp.concatenate(idx_parts)
    q = jnp.concatenate(q_parts, axis=0)

    encoding_indices = idx_all.reshape(n_tok, 1)
    quantized_st = jnp.transpose(q.reshape(B, H, W, C), (0, 3, 1, 2))
    loss = (1.0 + _COMMIT) * sum(loss_parts) / (B * C * H * W)
    return (quantized_st, loss, encoding_indices)


# speedup vs baseline: 1.1908x; 1.1908x over previous
import functools

import jax
import jax.numpy as jnp
from jax import lax
from jax.experimental import pallas as pl
from jax.experimental.pallas import tpu as pltpu
from jax.experimental.pallas import tpu_sc as plsc

_NUM_EMB = 8192
_DIM = 256
_COMMIT = 0.25
_T = 1024


def _dist_argmin_kernel(x_ref, e_ref, idx_ref, loss_ref, b_ref):
    i = pl.program_id(0)

    @pl.when(i == 0)
    def _():
        e0 = e_ref[...]
        b_ref[...] = lax.dot_general(
            jnp.ones((1, _DIM), jnp.float32), e0 * e0,
            (((1,), (1,)), ((), ())))
        loss_ref[0, 0] = 0.0

    x = x_ref[...]
    e = e_ref[...]

    a = jnp.sum(x * x, axis=1, keepdims=True)
    m = lax.dot_general(x, e, (((1,), (1,)), ((), ())))

    d = a + b_ref[0:1, :] - 2.0 * m

    dmin = jnp.min(d, axis=1, keepdims=True)
    iotaf = lax.broadcasted_iota(jnp.int32, d.shape, 1).astype(jnp.float32)
    idxf = jnp.min(jnp.where(d == dmin, iotaf, float(_NUM_EMB)), axis=1)
    idx_ref[...] = idxf.astype(jnp.int32)[None, None, :]
    loss_ref[0, 0] += jnp.sum(dmin)


@functools.lru_cache(maxsize=4)
def _make_sc_gather(n_rows):
    info = plsc.get_sparse_core_info()
    n_cores = info.num_cores
    rows_per_w = n_rows // (n_cores * info.num_subcores)

    def _sc_gather_kernel(table_hbm, idx_hbm, out_hbm, idx_v, rows_v, sem):
        wid = lax.axis_index("s") * n_cores + lax.axis_index("c")
        base = wid * rows_per_w
        pltpu.sync_copy(idx_hbm.at[pl.ds(base, rows_per_w)], idx_v)
        pltpu.async_copy(table_hbm.at[idx_v], rows_v, sem).wait()
        pltpu.sync_copy(rows_v, out_hbm.at[pl.ds(base, rows_per_w)])

    return functools.partial(
        pl.kernel,
        out_type=jax.ShapeDtypeStruct((n_rows, _DIM), jnp.float32),
        mesh=plsc.VectorSubcoreMesh(core_axis_name="c", subcore_axis_name="s"),
        scratch_types=[
            pltpu.VMEM((rows_per_w,), jnp.int32),
            pltpu.VMEM((rows_per_w, _DIM), jnp.float32),
            pltpu.SemaphoreType.DMA,
        ],
    )(_sc_gather_kernel)


_N_SPLIT = 1


def _dist_argmin(flat_chunk):
    n_tok = flat_chunk.shape[0]
    return pl.pallas_call(
        _dist_argmin_kernel,
        grid=(n_tok // _T,),
        in_specs=[
            pl.BlockSpec((_T, _DIM), lambda i: (i, 0)),
            pl.BlockSpec((_NUM_EMB, _DIM), lambda i: (0, 0)),
        ],
        out_specs=[
            pl.BlockSpec((1, 1, _T), lambda i: (i, 0, 0)),
            pl.BlockSpec(memory_space=pltpu.SMEM),
        ],
        out_shape=[
            jax.ShapeDtypeStruct((n_tok // _T, 1, _T), jnp.int32),
            jax.ShapeDtypeStruct((1, 1), jnp.float32),
        ],
        scratch_shapes=[pltpu.VMEM((1, _NUM_EMB), jnp.float32)],
    )


def kernel(inputs, emb_weight):
    B, C, H, W = inputs.shape
    n_tok = B * H * W
    n_chunk = n_tok // _N_SPLIT
    flat = jnp.transpose(inputs, (0, 2, 3, 1)).reshape(n_tok, _DIM)

    idx_parts, q_parts, loss_parts = [], [], []
    for s in range(_N_SPLIT):
        flat_s = lax.slice_in_dim(flat, s * n_chunk, (s + 1) * n_chunk)
        idx3, loss_sum = _dist_argmin(flat_s)(flat_s, emb_weight)
        idx_flat = idx3.reshape(n_chunk)
        q_parts.append(_make_sc_gather(n_chunk)(emb_weight, idx_flat))
        idx_parts.append(idx_flat)
        loss_parts.append(loss_sum[0, 0])

    idx_all = jnp.concatenate(idx_parts)
    q = jnp.concatenate(q_parts, axis=0)

    encoding_indices = idx_all.reshape(n_tok, 1)
    quantized_st = jnp.transpose(q.reshape(B, H, W, C), (0, 3, 1, 2))
    loss = (1.0 + _COMMIT) * sum(loss_parts) / (B * C * H * W)
    return (quantized_st, loss, encoding_indices)
